# baseline (speedup 1.0000x reference)
import functools

import jax
import jax.numpy as jnp
from jax import lax
from jax.experimental import pallas as pl
from jax.experimental.pallas import tpu as pltpu
from jax.experimental.pallas import tpu_sc as plsc

D = 128
NC = 2
NS = 16
NW = NC * NS

N_G, EH_G, N_T, EH_T, VOCAB = 10000, 160000, 5000, 5000, 800
N_ITERS = 4


def _pad_to(x, n, val=0):
    pad = [(0, n - x.shape[0])] + [(0, 0)] * (x.ndim - 1)
    return jnp.pad(x, pad, constant_values=val)


def _round_up(n, m):
    return (n + m - 1) // m * m


def _sc_gather(table, idx2d, out_rows, frac0=0.5):
    ch = idx2d.shape[1]
    n_tot = idx2d.shape[0]
    per_pair = n_tot // NS
    nb = min(4, per_pair // 2)
    m = max(nb, 8)
    if per_pair >= 2 * m and per_pair % m == 0:
        nc0 = min(max(int(per_pair * frac0) // m * m, m), per_pair - m)
    else:
        nc0 = per_pair // 2
    nc1 = per_pair - nc0
    mesh = plsc.VectorSubcoreMesh(core_axis_name="c", subcore_axis_name="s")

    @functools.partial(
        pl.kernel,
        out_type=jax.ShapeDtypeStruct((out_rows, D), jnp.float32),
        mesh=mesh,
        scratch_types=(
            [pltpu.VMEM((max(nc0, nc1), ch), jnp.int32)]
            + [pltpu.VMEM((ch, D), jnp.float32) for _ in range(nb)]
            + [pltpu.SemaphoreType.DMA for _ in range(2 * nb)]
        ),
    )
    def k(table_hbm, idx_hbm, out_hbm, idx_v, *rest):
        bufs, semi, semo = rest[:nb], rest[nb:2 * nb], rest[2 * nb:3 * nb]
        cid = lax.axis_index("c")
        sid = lax.axis_index("s")
        n_chunks = jnp.where(cid == 0, nc0, nc1)
        base_c = jnp.where(cid == 0, sid * nc0, NS * nc0 + sid * nc1)
        pltpu.sync_copy(
            idx_hbm.at[pl.ds(jnp.where(cid == 0, base_c, base_c - (nc0 - nc1)),
                             max(nc0, nc1))],
            idx_v)
        off = jnp.where(cid == 0, 0, nc0 - nc1)
        for b in range(nb):
            pltpu.async_copy(table_hbm.at[idx_v.at[off + b]], bufs[b], semi[b])

        @pl.loop(0, n_chunks, step=nb)
        def _(g):
            for b in range(nb):
                j = g + b
                pltpu.make_async_copy(
                    table_hbm.at[idx_v.at[off + j]], bufs[b], semi[b]).wait()
                pltpu.async_copy(
                    bufs[b], out_hbm.at[pl.ds((base_c + j) * ch, ch)], semo[b])
            for b in range(nb):
                j = g + b
                pltpu.make_async_copy(
                    bufs[b], out_hbm.at[pl.ds((base_c + j) * ch, ch)],
                    semo[b]).wait()

                @pl.when(j + nb < n_chunks)
                def _():
                    pltpu.async_copy(
                        table_hbm.at[idx_v.at[off + j + nb]], bufs[b], semi[b])

    return k(table, idx2d)


def _sc_pre_gather(ab_tab, u2d, v2d, ehp, frac0=0.5):
    ch = u2d.shape[1]
    per_pair = u2d.shape[0] // NS
    nb = min(2, per_pair // 2)
    m = max(nb, 8)
    if per_pair >= 2 * m and per_pair % m == 0:
        nc0 = min(max(int(per_pair * frac0) // m * m, m), per_pair - m)
    else:
        nc0 = per_pair // 2
    nc1 = per_pair - nc0
    n_chunks = max(nc0, nc1)
    mesh = plsc.VectorSubcoreMesh(core_axis_name="c", subcore_axis_name="s")

    @functools.partial(
        pl.kernel,
        out_type=jax.ShapeDtypeStruct((2 * ehp, D), jnp.float32),
        mesh=mesh,
        scratch_types=(
            [pltpu.VMEM((n_chunks, ch), jnp.int32) for _ in range(2)]
            + [pltpu.VMEM((ch, 2 * D), jnp.float32) for _ in range(2 * nb)]
            + [pltpu.VMEM((ch, D), jnp.float32) for _ in range(2 * nb)]
            + [pltpu.SemaphoreType.DMA for _ in range(3 * nb)]
        ),
    )
    def k(ab_hbm, u_hbm, v_hbm, out_hbm, u_v, v_v, *rest):
        abu = rest[:nb]
        abv = rest[nb:2 * nb]
        pf = rest[2 * nb:3 * nb]
        pb = rest[3 * nb:4 * nb]
        semu = rest[4 * nb:5 * nb]
        semv = rest[5 * nb:6 * nb]
        semo = rest[6 * nb:7 * nb]
        cid = lax.axis_index("c")
        sid = lax.axis_index("s")
        my_nc = jnp.where(cid == 0, nc0, nc1)
        base_c = jnp.where(cid == 0, sid * nc0, NS * nc0 + sid * nc1)
        stage = jnp.where(cid == 0, base_c, base_c - (nc0 - nc1))
        off = jnp.where(cid == 0, 0, nc0 - nc1)
        pltpu.sync_copy(u_hbm.at[pl.ds(stage, n_chunks)], u_v)
        pltpu.sync_copy(v_hbm.at[pl.ds(stage, n_chunks)], v_v)
        for b in range(nb):
            pltpu.async_copy(ab_hbm.at[u_v.at[off + b]], abu[b], semu[b])
            pltpu.async_copy(ab_hbm.at[v_v.at[off + b]], abv[b], semv[b])

        @pl.loop(0, my_nc, step=nb)
        def _(g):
            for b in range(nb):
                j = g + b
                pltpu.make_async_copy(
                    ab_hbm.at[u_v.at[off + j]], abu[b], semu[b]).wait()
                pltpu.make_async_copy(
                    ab_hbm.at[v_v.at[off + j]], abv[b], semv[b]).wait()

                @pl.loop(0, ch)
                def _(r):
                    for s in range(D // 16):
                        sa = pl.ds(s * 16, 16)
                        sb = pl.ds(D + s * 16, 16)
                        pf[b][r, sa] = abu[b][r, sa] + abv[b][r, sb]
                        pb[b][r, sa] = abv[b][r, sa] + abu[b][r, sb]

                pltpu.async_copy(
                    pf[b], out_hbm.at[pl.ds((base_c + j) * ch, ch)], semo[b])
                pltpu.async_copy(
                    pb[b], out_hbm.at[pl.ds(ehp + (base_c + j) * ch, ch)],
                    semo[b])
            for b in range(nb):
                j = g + b
                pltpu.make_async_copy(
                    pf[b], out_hbm.at[pl.ds((base_c + j) * ch, ch)],
                    semo[b]).wait()
                pltpu.make_async_copy(
                    pb[b], out_hbm.at[pl.ds(ehp + (base_c + j) * ch, ch)],
                    semo[b]).wait()

                @pl.when(j + nb < my_nc)
                def _():
                    pltpu.async_copy(
                        ab_hbm.at[u_v.at[off + j + nb]], abu[b], semu[b])
                    pltpu.async_copy(
                        ab_hbm.at[v_v.at[off + j + nb]], abv[b], semv[b])

    return k(ab_tab, u2d, v2d)


def _sc_scatter_add(data, idx2d, zeros, n_pad, row_off=0):
    ch = idx2d.shape[1]
    n_chunks = idx2d.shape[0] // NW
    nb = min(2, n_chunks)
    rpt = n_pad // NS
    mesh = plsc.VectorSubcoreMesh(core_axis_name="c", subcore_axis_name="s")

    @functools.partial(
        pl.kernel,
        out_type=jax.ShapeDtypeStruct((NC, n_pad, D), jnp.float32),
        mesh=mesh,
        scratch_types=(
            [pltpu.VMEM((n_chunks, ch), jnp.int32)]
            + [pltpu.VMEM((ch, D), jnp.float32) for _ in range(nb)]
            + [pltpu.VMEM_SHARED((n_pad, D), jnp.float32)]
            + [pltpu.SemaphoreType.DMA for _ in range(2 * nb)]
        ),
    )
    def k(data_hbm, idx_hbm, zeros_hbm, out_hbm, idx_v, *rest):
        bufs = rest[:nb]
        agg_s = rest[nb]
        semi, semo = rest[nb + 1:2 * nb + 1], rest[2 * nb + 1:3 * nb + 1]
        cid = lax.axis_index("c")
        sid = lax.axis_index("s")
        wid = sid * NC + cid
        base_c = wid * n_chunks
        my = pl.ds(sid * rpt, rpt)
        pltpu.sync_copy(zeros_hbm, agg_s.at[my])
        pltpu.sync_copy(idx_hbm.at[pl.ds(base_c, n_chunks)], idx_v)
        plsc.subcore_barrier()

        def in_cp(j, b, sem):
            return pltpu.make_async_copy(
                data_hbm.at[pl.ds(row_off + (base_c + j) * ch, ch)],
                bufs[b], sem)

        for b in range(nb):
            in_cp(b, b, semi[b]).start()

        @pl.loop(0, n_chunks, step=nb)
        def _(g):
            for b in range(nb):
                j = g + b
                in_cp(j, b, semi[b]).wait()
                pltpu.async_copy(bufs[b], agg_s.at[idx_v.at[j]], semo[b],
                                 add=True)
            for b in range(nb):
                j = g + b
                pltpu.make_async_copy(
                    bufs[b], agg_s.at[idx_v.at[j]], semo[b]).wait()

                @pl.when(j + nb < n_chunks)
                def _():
                    in_cp(j + nb, b, semi[b]).start()

        plsc.subcore_barrier()
        pltpu.sync_copy(agg_s.at[my], out_hbm.at[cid, my])

    return k(data, idx2d, zeros)


def _tc_node_ab(f_pad, w1, w2, b1):
    np_, _ = f_pad.shape
    blk = np_ // 16

    def body(f_ref, w1_ref, w2_ref, b1_ref, ab_ref):
        f = f_ref[...]
        ab_ref[:, :D] = f @ w1_ref[...] + b1_ref[...]
        ab_ref[:, D:] = f @ w2_ref[...]

    return pl.pallas_call(
        body,
        grid=(16,),
        in_specs=[
            pl.BlockSpec((blk, D), lambda i: (i, 0)),
            pl.BlockSpec((D, D), lambda i: (0, 0)),
            pl.BlockSpec((D, D), lambda i: (0, 0)),
            pl.BlockSpec((1, D), lambda i: (0, 0)),
        ],
        out_specs=pl.BlockSpec((blk, 2 * D), lambda i: (i, 0)),
        out_shape=jax.ShapeDtypeStruct((np_, 2 * D), jnp.float32),
    )(f_pad, w1, w2, b1.reshape(1, D))


def _tc_relu(x):
    n = x.shape[0]
    blk = 4096
    grid = n // blk

    def body(x_ref, o_ref):
        o_ref[...] = jnp.maximum(x_ref[...], 0.0)

    return pl.pallas_call(
        body,
        grid=(grid,),
        in_specs=[pl.BlockSpec((blk, D), lambda i: (i, 0))],
        out_specs=pl.BlockSpec((blk, D), lambda i: (i, 0)),
        out_shape=jax.ShapeDtypeStruct((n, D), jnp.float32),
    )(x)


def _tc_combine(partials):
    np_ = partials.shape[1]
    blk = np_ // 16

    def body(p_ref, o_ref):
        o_ref[...] = p_ref[0] + p_ref[1]

    return pl.pallas_call(
        body,
        grid=(16,),
        in_specs=[pl.BlockSpec((2, blk, D), lambda i: (0, i, 0))],
        out_specs=pl.BlockSpec((blk, D), lambda i: (i, 0)),
        out_shape=jax.ShapeDtypeStruct((np_, D), jnp.float32),
    )(partials)


def _tc_msg_update(gathered, msg_old, pre, w3):
    ep = gathered.shape[0]
    blk = 1024
    nb = ep // blk
    nbh = nb // 2

    def body(g_ref, mr_ref, p_ref, w3_ref, o_ref):
        s = g_ref[...] - mr_ref[...]
        o_ref[...] = jnp.maximum(p_ref[...] + s @ w3_ref[...], 0.0)

    return pl.pallas_call(
        body,
        grid=(nb,),
        in_specs=[
            pl.BlockSpec((blk, D), lambda i: (i, 0)),
            pl.BlockSpec((blk, D), lambda i: ((i + nbh) % nb, 0)),
            pl.BlockSpec((blk, D), lambda i: (i, 0)),
            pl.BlockSpec((D, D), lambda i: (0, 0)),
        ],
        out_specs=pl.BlockSpec((blk, D), lambda i: (i, 0)),
        out_shape=jax.ShapeDtypeStruct((ep, D), jnp.float32),
    )(gathered, msg_old, pre, w3)


def _tc_readout(f, ns, u1, u2, b2):
    n = f.shape[0]
    blk = 1000
    grid = n // blk

    def body(f_ref, ns_ref, u1_ref, u2_ref, b2_ref, o_ref):
        o_ref[...] = jnp.maximum(
            f_ref[...] @ u1_ref[...] + ns_ref[...] @ u2_ref[...] + b2_ref[...],
            0.0)

    return pl.pallas_call(
        body,
        grid=(grid,),
        in_specs=[
            pl.BlockSpec((blk, D), lambda i: (i, 0)),
            pl.BlockSpec((blk, D), lambda i: (i, 0)),
            pl.BlockSpec((D, D), lambda i: (0, 0)),
            pl.BlockSpec((D, D), lambda i: (0, 0)),
            pl.BlockSpec((1, D), lambda i: (0, 0)),
        ],
        out_specs=pl.BlockSpec((blk, D), lambda i: (i, 0)),
        out_shape=jax.ShapeDtypeStruct((n, D), jnp.float32),
    )(f, ns, u1, u2, b2.reshape(1, D))


def _encode(f, u, v, w1, w2, w3, b1, u1, u2, b2, n, n_iters):
    eh = u.shape[0]
    ehp = _round_up(eh, NW * 128)
    ep = 2 * ehp
    n_pad = _round_up(n + 1, NS * 8)
    rpt = n_pad // NS

    u_p = _pad_to(u, ehp, n)
    v_p = _pad_to(v, ehp, n)
    src = jnp.concatenate([u_p, v_p]).reshape(ep // 128, 128)
    dst = jnp.concatenate([v_p, u_p]).reshape(ep // 64, 64)
    u2d = u_p.reshape(ehp // 32, 32)
    v2d = v_p.reshape(ehp // 32, 32)
    f_pad = _pad_to(f, n_pad)
    zeros = jnp.zeros((rpt, D), jnp.float32)

    ab_tab = _tc_node_ab(f_pad, w1, w2, b1)
    pre = _sc_pre_gather(ab_tab, u2d, v2d, ehp, frac0=0.62)
    msg = _tc_relu(pre)
    for _ in range(n_iters - 1):
        partials = _sc_scatter_add(msg, dst, zeros, n_pad)
        agg = _tc_combine(partials)
        gathered = _sc_gather(agg, src, ep, frac0=0.8)
        msg = _tc_msg_update(gathered, msg, pre, w3)
    partials = _sc_scatter_add(msg, dst, zeros, n_pad)
    ns = _tc_combine(partials)[:n]
    return _tc_readout(f, ns, u1, u2, b2)


def kernel(f_G, u_G, v_G, id_T, u_T, v_T, embeddings, W1_G, W2_G, W3_G, b1_G,
           U1_G, U2_G, b2_G, W1_T, W2_T, W3_T, b1_T, U1_T, U2_T, b2_T):
    x_G = _encode(f_G, u_G, v_G, W1_G, W2_G, W3_G, b1_G, U1_G, U2_G, b2_G,
                  N_G, N_ITERS)
    idp = _round_up(N_T, NW * 128)
    id2d = _pad_to(id_T, idp).reshape(idp // 128, 128)
    f_T = _sc_gather(embeddings, id2d, idp)[:N_T]
    x_T = _encode(f_T, u_T, v_T, W1_T, W2_T, W3_T, b1_T, U1_T, U2_T, b2_T,
                  N_T, N_ITERS)
    return x_G, x_T


# speedup vs baseline: 1.0154x; 1.0154x over previous
import functools

import jax
import jax.numpy as jnp
from jax import lax
from jax.experimental import pallas as pl
from jax.experimental.pallas import tpu as pltpu
from jax.experimental.pallas import tpu_sc as plsc

D = 128
NC = 2
NS = 16
NW = NC * NS

N_G, EH_G, N_T, EH_T, VOCAB = 10000, 160000, 5000, 5000, 800
N_ITERS = 4


def _pad_to(x, n, val=0):
    pad = [(0, n - x.shape[0])] + [(0, 0)] * (x.ndim - 1)
    return jnp.pad(x, pad, constant_values=val)


def _round_up(n, m):
    return (n + m - 1) // m * m


def _sc_gather(table, idx2d, out_rows, frac0=0.5):
    ch = idx2d.shape[1]
    n_tot = idx2d.shape[0]
    per_pair = n_tot // NS
    nb = min(4, per_pair // 2)
    m = max(nb, 8)
    if per_pair >= 2 * m and per_pair % m == 0:
        nc0 = min(max(-(-int(per_pair * frac0) // m) * m,
                      -(-per_pair // (2 * m)) * m), per_pair - m)
    else:
        nc0 = per_pair // 2
    nc1 = per_pair - nc0
    mesh = plsc.VectorSubcoreMesh(core_axis_name="c", subcore_axis_name="s")

    @functools.partial(
        pl.kernel,
        out_type=jax.ShapeDtypeStruct((out_rows, D), jnp.float32),
        mesh=mesh,
        scratch_types=(
            [pltpu.VMEM((max(nc0, nc1), ch), jnp.int32)]
            + [pltpu.VMEM((ch, D), jnp.float32) for _ in range(nb)]
            + [pltpu.SemaphoreType.DMA for _ in range(2 * nb)]
        ),
    )
    def k(table_hbm, idx_hbm, out_hbm, idx_v, *rest):
        bufs, semi, semo = rest[:nb], rest[nb:2 * nb], rest[2 * nb:3 * nb]
        cid = lax.axis_index("c")
        sid = lax.axis_index("s")
        n_chunks = jnp.where(cid == 0, nc0, nc1)
        base_c = jnp.where(cid == 0, sid * nc0, NS * nc0 + sid * nc1)
        pltpu.sync_copy(
            idx_hbm.at[pl.ds(jnp.where(cid == 0, base_c, base_c - (nc0 - nc1)),
                             max(nc0, nc1))],
            idx_v)
        off = jnp.where(cid == 0, 0, nc0 - nc1)
        for b in range(nb):
            pltpu.async_copy(table_hbm.at[idx_v.at[off + b]], bufs[b], semi[b])

        @pl.loop(0, n_chunks, step=nb)
        def _(g):
            for b in range(nb):
                j = g + b
                pltpu.make_async_copy(
                    table_hbm.at[idx_v.at[off + j]], bufs[b], semi[b]).wait()
                pltpu.async_copy(
                    bufs[b], out_hbm.at[pl.ds((base_c + j) * ch, ch)], semo[b])
            for b in range(nb):
                j = g + b
                pltpu.make_async_copy(
                    bufs[b], out_hbm.at[pl.ds((base_c + j) * ch, ch)],
                    semo[b]).wait()

                @pl.when(j + nb < n_chunks)
                def _():
                    pltpu.async_copy(
                        table_hbm.at[idx_v.at[off + j + nb]], bufs[b], semi[b])

    return k(table, idx2d)


def _sc_pre_gather(ab_tab, u2d, v2d, ehp, frac0=0.5):
    ch = u2d.shape[1]
    per_pair = u2d.shape[0] // NS
    nb = min(2, per_pair // 2)
    m = max(nb, 8)
    if per_pair >= 2 * m and per_pair % m == 0:
        nc0 = min(max(-(-int(per_pair * frac0) // m) * m,
                      -(-per_pair // (2 * m)) * m), per_pair - m)
    else:
        nc0 = per_pair // 2
    nc1 = per_pair - nc0
    n_chunks = max(nc0, nc1)
    mesh = plsc.VectorSubcoreMesh(core_axis_name="c", subcore_axis_name="s")

    @functools.partial(
        pl.kernel,
        out_type=jax.ShapeDtypeStruct((2 * ehp, D), jnp.float32),
        mesh=mesh,
        scratch_types=(
            [pltpu.VMEM((n_chunks, ch), jnp.int32) for _ in range(2)]
            + [pltpu.VMEM((ch, 2 * D), jnp.float32) for _ in range(2 * nb)]
            + [pltpu.VMEM((ch, D), jnp.float32) for _ in range(2 * nb)]
            + [pltpu.SemaphoreType.DMA for _ in range(3 * nb)]
        ),
    )
    def k(ab_hbm, u_hbm, v_hbm, out_hbm, u_v, v_v, *rest):
        abu = rest[:nb]
        abv = rest[nb:2 * nb]
        pf = rest[2 * nb:3 * nb]
        pb = rest[3 * nb:4 * nb]
        semu = rest[4 * nb:5 * nb]
        semv = rest[5 * nb:6 * nb]
        semo = rest[6 * nb:7 * nb]
        cid = lax.axis_index("c")
        sid = lax.axis_index("s")
        my_nc = jnp.where(cid == 0, nc0, nc1)
        base_c = jnp.where(cid == 0, sid * nc0, NS * nc0 + sid * nc1)
        stage = jnp.where(cid == 0, base_c, base_c - (nc0 - nc1))
        off = jnp.where(cid == 0, 0, nc0 - nc1)
        pltpu.sync_copy(u_hbm.at[pl.ds(stage, n_chunks)], u_v)
        pltpu.sync_copy(v_hbm.at[pl.ds(stage, n_chunks)], v_v)
        for b in range(nb):
            pltpu.async_copy(ab_hbm.at[u_v.at[off + b]], abu[b], semu[b])
            pltpu.async_copy(ab_hbm.at[v_v.at[off + b]], abv[b], semv[b])

        @pl.loop(0, my_nc, step=nb)
        def _(g):
            for b in range(nb):
                j = g + b
                pltpu.make_async_copy(
                    ab_hbm.at[u_v.at[off + j]], abu[b], semu[b]).wait()
                pltpu.make_async_copy(
                    ab_hbm.at[v_v.at[off + j]], abv[b], semv[b]).wait()

                @pl.loop(0, ch)
                def _(r):
                    for s in range(D // 16):
                        sa = pl.ds(s * 16, 16)
                        sb = pl.ds(D + s * 16, 16)
                        pf[b][r, sa] = abu[b][r, sa] + abv[b][r, sb]
                        pb[b][r, sa] = abv[b][r, sa] + abu[b][r, sb]

                pltpu.async_copy(
                    pf[b], out_hbm.at[pl.ds((base_c + j) * ch, ch)], semo[b])
                pltpu.async_copy(
                    pb[b], out_hbm.at[pl.ds(ehp + (base_c + j) * ch, ch)],
                    semo[b])
            for b in range(nb):
                j = g + b
                pltpu.make_async_copy(
                    pf[b], out_hbm.at[pl.ds((base_c + j) * ch, ch)],
                    semo[b]).wait()
                pltpu.make_async_copy(
                    pb[b], out_hbm.at[pl.ds(ehp + (base_c + j) * ch, ch)],
                    semo[b]).wait()

                @pl.when(j + nb < my_nc)
                def _():
                    pltpu.async_copy(
                        ab_hbm.at[u_v.at[off + j + nb]], abu[b], semu[b])
                    pltpu.async_copy(
                        ab_hbm.at[v_v.at[off + j + nb]], abv[b], semv[b])

    return k(ab_tab, u2d, v2d)


def _sc_scatter_add(data, idx2d, zeros, n_pad, row_off=0):
    ch = idx2d.shape[1]
    n_chunks = idx2d.shape[0] // NW
    nb = min(2, n_chunks)
    rpt = n_pad // NS
    mesh = plsc.VectorSubcoreMesh(core_axis_name="c", subcore_axis_name="s")

    @functools.partial(
        pl.kernel,
        out_type=jax.ShapeDtypeStruct((NC, n_pad, D), jnp.float32),
        mesh=mesh,
        scratch_types=(
            [pltpu.VMEM((n_chunks, ch), jnp.int32)]
            + [pltpu.VMEM((ch, D), jnp.float32) for _ in range(nb)]
            + [pltpu.VMEM_SHARED((n_pad, D), jnp.float32)]
            + [pltpu.SemaphoreType.DMA for _ in range(2 * nb)]
        ),
    )
    def k(data_hbm, idx_hbm, zeros_hbm, out_hbm, idx_v, *rest):
        bufs = rest[:nb]
        agg_s = rest[nb]
        semi, semo = rest[nb + 1:2 * nb + 1], rest[2 * nb + 1:3 * nb + 1]
        cid = lax.axis_index("c")
        sid = lax.axis_index("s")
        wid = sid * NC + cid
        base_c = wid * n_chunks
        my = pl.ds(sid * rpt, rpt)
        pltpu.sync_copy(zeros_hbm, agg_s.at[my])
        pltpu.sync_copy(idx_hbm.at[pl.ds(base_c, n_chunks)], idx_v)
        plsc.subcore_barrier()

        def in_cp(j, b, sem):
            return pltpu.make_async_copy(
                data_hbm.at[pl.ds(row_off + (base_c + j) * ch, ch)],
                bufs[b], sem)

        for b in range(nb):
            in_cp(b, b, semi[b]).start()

        @pl.loop(0, n_chunks, step=nb)
        def _(g):
            for b in range(nb):
                j = g + b
                in_cp(j, b, semi[b]).wait()
                pltpu.async_copy(bufs[b], agg_s.at[idx_v.at[j]], semo[b],
                                 add=True)
            for b in range(nb):
                j = g + b
                pltpu.make_async_copy(
                    bufs[b], agg_s.at[idx_v.at[j]], semo[b]).wait()

                @pl.when(j + nb < n_chunks)
                def _():
                    in_cp(j + nb, b, semi[b]).start()

        plsc.subcore_barrier()
        pltpu.sync_copy(agg_s.at[my], out_hbm.at[cid, my])

    return k(data, idx2d, zeros)


def _tc_node_ab(f_pad, w1, w2, b1):
    np_, _ = f_pad.shape
    blk = np_ // 16

    def body(f_ref, w1_ref, w2_ref, b1_ref, ab_ref):
        f = f_ref[...]
        ab_ref[:, :D] = f @ w1_ref[...] + b1_ref[...]
        ab_ref[:, D:] = f @ w2_ref[...]

    return pl.pallas_call(
        body,
        grid=(16,),
        in_specs=[
            pl.BlockSpec((blk, D), lambda i: (i, 0)),
            pl.BlockSpec((D, D), lambda i: (0, 0)),
            pl.BlockSpec((D, D), lambda i: (0, 0)),
            pl.BlockSpec((1, D), lambda i: (0, 0)),
        ],
        out_specs=pl.BlockSpec((blk, 2 * D), lambda i: (i, 0)),
        out_shape=jax.ShapeDtypeStruct((np_, 2 * D), jnp.float32),
    )(f_pad, w1, w2, b1.reshape(1, D))


def _tc_relu(x):
    n = x.shape[0]
    blk = 4096
    grid = n // blk

    def body(x_ref, o_ref):
        o_ref[...] = jnp.maximum(x_ref[...], 0.0)

    return pl.pallas_call(
        body,
        grid=(grid,),
        in_specs=[pl.BlockSpec((blk, D), lambda i: (i, 0))],
        out_specs=pl.BlockSpec((blk, D), lambda i: (i, 0)),
        out_shape=jax.ShapeDtypeStruct((n, D), jnp.float32),
    )(x)


def _tc_combine(partials):
    np_ = partials.shape[1]
    blk = np_ // 16

    def body(p_ref, o_ref):
        o_ref[...] = p_ref[0] + p_ref[1]

    return pl.pallas_call(
        body,
        grid=(16,),
        in_specs=[pl.BlockSpec((2, blk, D), lambda i: (0, i, 0))],
        out_specs=pl.BlockSpec((blk, D), lambda i: (i, 0)),
        out_shape=jax.ShapeDtypeStruct((np_, D), jnp.float32),
    )(partials)


def _tc_msg_update(gathered, msg_old, pre, w3, blk_off=0):
    ep = msg_old.shape[0]
    blk = 1024
    nb = ep // blk
    nbh = nb // 2

    def body(g_ref, mr_ref, p_ref, w3_ref, o_ref):
        s = g_ref[...] - mr_ref[...]
        o_ref[...] = jnp.maximum(p_ref[...] + s @ w3_ref[...], 0.0)

    return pl.pallas_call(
        body,
        grid=(nb,),
        in_specs=[
            pl.BlockSpec((blk, D), lambda i: (blk_off + i, 0)),
            pl.BlockSpec((blk, D), lambda i: ((i + nbh) % nb, 0)),
            pl.BlockSpec((blk, D), lambda i: (i, 0)),
            pl.BlockSpec((D, D), lambda i: (0, 0)),
        ],
        out_specs=pl.BlockSpec((blk, D), lambda i: (i, 0)),
        out_shape=jax.ShapeDtypeStruct((ep, D), jnp.float32),
    )(gathered, msg_old, pre, w3)


def _tc_embw(emb, w1, w2, b1):
    def body(e_ref, w1_ref, w2_ref, b1_ref, o_ref):
        e = e_ref[...]
        o_ref[:, :D] = e @ w1_ref[...] + b1_ref[...]
        o_ref[:, D:2 * D] = e @ w2_ref[...]
        o_ref[:, 2 * D:] = e

    return pl.pallas_call(
        body,
        grid=(1,),
        in_specs=[
            pl.BlockSpec((VOCAB, D), lambda i: (0, 0)),
            pl.BlockSpec((D, D), lambda i: (0, 0)),
            pl.BlockSpec((D, D), lambda i: (0, 0)),
            pl.BlockSpec((1, D), lambda i: (0, 0)),
        ],
        out_specs=pl.BlockSpec((VOCAB, 3 * D), lambda i: (0, 0)),
        out_shape=jax.ShapeDtypeStruct((VOCAB, 3 * D), jnp.float32),
    )(emb, w1, w2, b1.reshape(1, D))


def _tc_onehot_rows(idx_col, table):
    n, w = idx_col.shape[0], table.shape[1]
    v = table.shape[0]
    blk = 640
    grid = n // blk

    def body(i_ref, t_ref, o_ref):
        ids = i_ref[...]
        io = lax.broadcasted_iota(jnp.int32, (1, v), 1)
        m = (ids == io).astype(jnp.float32)
        o_ref[...] = m @ t_ref[...]

    return pl.pallas_call(
        body,
        grid=(grid,),
        in_specs=[
            pl.BlockSpec((blk, 1), lambda i: (i, 0)),
            pl.BlockSpec((v, w), lambda i: (0, 0)),
        ],
        out_specs=pl.BlockSpec((blk, w), lambda i: (i, 0)),
        out_shape=jax.ShapeDtypeStruct((n, w), jnp.float32),
    )(idx_col, table)


def _tc_scatter_onehot(msg, dst3, n_pad):
    ep = msg.shape[0]
    eblk = 2048
    nblk = 640
    ne, nn = ep // eblk, n_pad // nblk

    def body(d_ref, m_ref, o_ref):
        n_i = pl.program_id(0)
        e_i = pl.program_id(1)
        ids = n_i * nblk + lax.broadcasted_iota(jnp.int32, (nblk, 1), 0)
        mask = (ids == d_ref[0]).astype(jnp.float32)
        contrib = mask @ m_ref[...]

        @pl.when(e_i == 0)
        def _():
            o_ref[...] = contrib

        @pl.when(e_i > 0)
        def _():
            o_ref[...] += contrib

    return pl.pallas_call(
        body,
        grid=(nn, ne),
        in_specs=[
            pl.BlockSpec((1, 1, eblk), lambda n, e: (e, 0, 0)),
            pl.BlockSpec((eblk, D), lambda n, e: (e, 0)),
        ],
        out_specs=pl.BlockSpec((nblk, D), lambda n, e: (n, 0)),
        out_shape=jax.ShapeDtypeStruct((n_pad, D), jnp.float32),
    )(dst3, msg)


def _tc_combine2(partials_g, agg_t):
    npg, npt = partials_g.shape[1], agg_t.shape[0]
    blk = 128
    nbg, nbt = npg // blk, npt // blk

    def body(p_ref, t_ref, o_ref):
        i = pl.program_id(0)

        @pl.when(i < nbg)
        def _():
            o_ref[...] = p_ref[0] + p_ref[1]

        @pl.when(i >= nbg)
        def _():
            o_ref[...] = t_ref[...]

    return pl.pallas_call(
        body,
        grid=(nbg + nbt,),
        in_specs=[
            pl.BlockSpec((2, blk, D),
                         lambda i: (0, jnp.minimum(i, nbg - 1), 0)),
            pl.BlockSpec((blk, D), lambda i: (jnp.maximum(i - nbg, 0), 0)),
        ],
        out_specs=pl.BlockSpec((blk, D), lambda i: (i, 0)),
        out_shape=jax.ShapeDtypeStruct((npg + npt, D), jnp.float32),
    )(partials_g, agg_t)


def _tc_readout(f, ns, u1, u2, b2):
    n = f.shape[0]
    blk = 1000
    grid = n // blk

    def body(f_ref, ns_ref, u1_ref, u2_ref, b2_ref, o_ref):
        o_ref[...] = jnp.maximum(
            f_ref[...] @ u1_ref[...] + ns_ref[...] @ u2_ref[...] + b2_ref[...],
            0.0)

    return pl.pallas_call(
        body,
        grid=(grid,),
        in_specs=[
            pl.BlockSpec((blk, D), lambda i: (i, 0)),
            pl.BlockSpec((blk, D), lambda i: (i, 0)),
            pl.BlockSpec((D, D), lambda i: (0, 0)),
            pl.BlockSpec((D, D), lambda i: (0, 0)),
            pl.BlockSpec((1, D), lambda i: (0, 0)),
        ],
        out_specs=pl.BlockSpec((blk, D), lambda i: (i, 0)),
        out_shape=jax.ShapeDtypeStruct((n, D), jnp.float32),
    )(f, ns, u1, u2, b2.reshape(1, D))


EHP_G = _round_up(EH_G, NW * 128)
EP_G = 2 * EHP_G
NP_G = _round_up(N_G + 1, NS * 8)
EHP_T = _round_up(EH_T, NW * 128)
EP_T = 2 * EHP_T
NP_T = _round_up(N_T + 1, NS * 8)
EP_CAT = EP_G + EP_T


def kernel(f_G, u_G, v_G, id_T, u_T, v_T, embeddings, W1_G, W2_G, W3_G, b1_G,
           U1_G, U2_G, b2_G, W1_T, W2_T, W3_T, b1_T, U1_T, U2_T, b2_T):
    u_gp = _pad_to(u_G, EHP_G, N_G)
    v_gp = _pad_to(v_G, EHP_G, N_G)
    u_tp = _pad_to(u_T, EHP_T, N_T)
    v_tp = _pad_to(v_T, EHP_T, N_T)
    dst_g = jnp.concatenate([v_gp, u_gp]).reshape(EP_G // 64, 64)
    dst_t3 = jnp.concatenate([v_tp, u_tp]).reshape(EP_T // 2048, 1, 2048)
    src_cat = jnp.concatenate(
        [u_gp, v_gp, NP_G + u_tp, NP_G + v_tp]).reshape(EP_CAT // 128, 128)
    u2d_g = u_gp.reshape(EHP_G // 32, 32)
    v2d_g = v_gp.reshape(EHP_G // 32, 32)
    u2d_t = u_tp.reshape(EHP_T // 32, 32)
    v2d_t = v_tp.reshape(EHP_T // 32, 32)
    id_col = _pad_to(id_T, NP_T).reshape(NP_T, 1)
    f_gpad = _pad_to(f_G, NP_G)
    zeros_g = jnp.zeros((NP_G // NS, D), jnp.float32)

    ab_g = _tc_node_ab(f_gpad, W1_G, W2_G, b1_G)
    pre_g = _sc_pre_gather(ab_g, u2d_g, v2d_g, EHP_G)
    embt = _tc_embw(embeddings, W1_T, W2_T, b1_T)
    fab_t = _tc_onehot_rows(id_col, embt)
    ab_t = fab_t[:, :2 * D]
    f_t = fab_t[:N_T, 2 * D:]
    pre_t = _sc_pre_gather(ab_t, u2d_t, v2d_t, EHP_T)

    msg_g = _tc_relu(pre_g)
    msg_t = _tc_relu(pre_t)
    for _ in range(N_ITERS - 1):
        partials_g = _sc_scatter_add(msg_g, dst_g, zeros_g, NP_G)
        agg_t = _tc_scatter_onehot(msg_t, dst_t3, NP_T)
        table = _tc_combine2(partials_g, agg_t)
        gathered = _sc_gather(table, src_cat, EP_CAT)
        msg_g = _tc_msg_update(gathered, msg_g, pre_g, W3_G)
        msg_t = _tc_msg_update(gathered, msg_t, pre_t, W3_T,
                               blk_off=EP_G // 1024)

    partials_g = _sc_scatter_add(msg_g, dst_g, zeros_g, NP_G)
    ns_g = _tc_combine(partials_g)[:N_G]
    ns_t = _tc_scatter_onehot(msg_t, dst_t3, NP_T)[:N_T]
    x_G = _tc_readout(f_G, ns_g, U1_G, U2_G, b2_G)
    x_T = _tc_readout(f_t, ns_t, U1_T, U2_T, b2_T)
    return x_G, x_T


# speedup vs baseline: 1.1294x; 1.1123x over previous
import functools

import jax
import jax.numpy as jnp
from jax import lax
from jax.experimental import pallas as pl
from jax.experimental.pallas import tpu as pltpu
from jax.experimental.pallas import tpu_sc as plsc

D = 128
NC = 2
NS = 16
NW = NC * NS

N_G, EH_G, N_T, EH_T, VOCAB = 10000, 160000, 5000, 5000, 800
N_ITERS = 4


def _pad_to(x, n, val=0):
    pad = [(0, n - x.shape[0])] + [(0, 0)] * (x.ndim - 1)
    return jnp.pad(x, pad, constant_values=val)


def _round_up(n, m):
    return (n + m - 1) // m * m


def _sc_gather(table, idx2d, out_rows, frac0=0.5):
    dt = table.dtype
    w = table.shape[1]
    ch = idx2d.shape[1]
    n_tot = idx2d.shape[0]
    per_pair = n_tot // NS
    nb = min(4, per_pair // 2)
    m = max(nb, 8)
    if per_pair >= 2 * m and per_pair % m == 0:
        nc0 = min(max(-(-int(per_pair * frac0) // m) * m,
                      -(-per_pair // (2 * m)) * m), per_pair - m)
    else:
        nc0 = per_pair // 2
    nc1 = per_pair - nc0
    mesh = plsc.VectorSubcoreMesh(core_axis_name="c", subcore_axis_name="s")

    @functools.partial(
        pl.kernel,
        out_type=jax.ShapeDtypeStruct((out_rows, w), dt),
        mesh=mesh,
        scratch_types=(
            [pltpu.VMEM((max(nc0, nc1), ch), jnp.int32)]
            + [pltpu.VMEM((ch, w), dt) for _ in range(nb)]
            + [pltpu.SemaphoreType.DMA for _ in range(2 * nb)]
        ),
    )
    def k(table_hbm, idx_hbm, out_hbm, idx_v, *rest):
        bufs, semi, semo = rest[:nb], rest[nb:2 * nb], rest[2 * nb:3 * nb]
        cid = lax.axis_index("c")
        sid = lax.axis_index("s")
        n_chunks = jnp.where(cid == 0, nc0, nc1)
        base_c = jnp.where(cid == 0, sid * nc0, NS * nc0 + sid * nc1)
        pltpu.sync_copy(
            idx_hbm.at[pl.ds(jnp.where(cid == 0, base_c, base_c - (nc0 - nc1)),
                             max(nc0, nc1))],
            idx_v)
        off = jnp.where(cid == 0, 0, nc0 - nc1)
        for b in range(nb):
            pltpu.async_copy(table_hbm.at[idx_v.at[off + b]], bufs[b], semi[b])

        @pl.loop(0, n_chunks, step=nb)
        def _(g):
            for b in range(nb):
                j = g + b
                pltpu.make_async_copy(
                    table_hbm.at[idx_v.at[off + j]], bufs[b], semi[b]).wait()
                pltpu.async_copy(
                    bufs[b], out_hbm.at[pl.ds((base_c + j) * ch, ch)], semo[b])
            for b in range(nb):
                j = g + b
                pltpu.make_async_copy(
                    bufs[b], out_hbm.at[pl.ds((base_c + j) * ch, ch)],
                    semo[b]).wait()

                @pl.when(j + nb < n_chunks)
                def _():
                    pltpu.async_copy(
                        table_hbm.at[idx_v.at[off + j + nb]], bufs[b], semi[b])

    return k(table, idx2d)


def _sc_pre_gather(ab_tab, u2d, v2d, ehp, frac0=0.5):
    ch = u2d.shape[1]
    per_pair = u2d.shape[0] // NS
    nb = min(2, per_pair // 2)
    m = max(nb, 8)
    if per_pair >= 2 * m and per_pair % m == 0:
        nc0 = min(max(-(-int(per_pair * frac0) // m) * m,
                      -(-per_pair // (2 * m)) * m), per_pair - m)
    else:
        nc0 = per_pair // 2
    nc1 = per_pair - nc0
    n_chunks = max(nc0, nc1)
    mesh = plsc.VectorSubcoreMesh(core_axis_name="c", subcore_axis_name="s")

    @functools.partial(
        pl.kernel,
        out_type=jax.ShapeDtypeStruct((2 * ehp, D), jnp.float32),
        mesh=mesh,
        scratch_types=(
            [pltpu.VMEM((n_chunks, ch), jnp.int32) for _ in range(2)]
            + [pltpu.VMEM((ch, 2 * D), jnp.float32) for _ in range(2 * nb)]
            + [pltpu.VMEM((ch, D), jnp.float32) for _ in range(2 * nb)]
            + [pltpu.SemaphoreType.DMA for _ in range(3 * nb)]
        ),
    )
    def k(ab_hbm, u_hbm, v_hbm, out_hbm, u_v, v_v, *rest):
        abu = rest[:nb]
        abv = rest[nb:2 * nb]
        pf = rest[2 * nb:3 * nb]
        pb = rest[3 * nb:4 * nb]
        semu = rest[4 * nb:5 * nb]
        semv = rest[5 * nb:6 * nb]
        semo = rest[6 * nb:7 * nb]
        cid = lax.axis_index("c")
        sid = lax.axis_index("s")
        my_nc = jnp.where(cid == 0, nc0, nc1)
        base_c = jnp.where(cid == 0, sid * nc0, NS * nc0 + sid * nc1)
        stage = jnp.where(cid == 0, base_c, base_c - (nc0 - nc1))
        off = jnp.where(cid == 0, 0, nc0 - nc1)
        pltpu.sync_copy(u_hbm.at[pl.ds(stage, n_chunks)], u_v)
        pltpu.sync_copy(v_hbm.at[pl.ds(stage, n_chunks)], v_v)
        for b in range(nb):
            pltpu.async_copy(ab_hbm.at[u_v.at[off + b]], abu[b], semu[b])
            pltpu.async_copy(ab_hbm.at[v_v.at[off + b]], abv[b], semv[b])

        @pl.loop(0, my_nc, step=nb)
        def _(g):
            for b in range(nb):
                j = g + b
                pltpu.make_async_copy(
                    ab_hbm.at[u_v.at[off + j]], abu[b], semu[b]).wait()
                pltpu.make_async_copy(
                    ab_hbm.at[v_v.at[off + j]], abv[b], semv[b]).wait()

                @pl.loop(0, ch)
                def _(r):
                    for s in range(D // 16):
                        sa = pl.ds(s * 16, 16)
                        sb = pl.ds(D + s * 16, 16)
                        pf[b][r, sa] = abu[b][r, sa] + abv[b][r, sb]
                        pb[b][r, sa] = abv[b][r, sa] + abu[b][r, sb]

                pltpu.async_copy(
                    pf[b], out_hbm.at[pl.ds((base_c + j) * ch, ch)], semo[b])
                pltpu.async_copy(
                    pb[b], out_hbm.at[pl.ds(ehp + (base_c + j) * ch, ch)],
                    semo[b])
            for b in range(nb):
                j = g + b
                pltpu.make_async_copy(
                    pf[b], out_hbm.at[pl.ds((base_c + j) * ch, ch)],
                    semo[b]).wait()
                pltpu.make_async_copy(
                    pb[b], out_hbm.at[pl.ds(ehp + (base_c + j) * ch, ch)],
                    semo[b]).wait()

                @pl.when(j + nb < my_nc)
                def _():
                    pltpu.async_copy(
                        ab_hbm.at[u_v.at[off + j + nb]], abu[b], semu[b])
                    pltpu.async_copy(
                        ab_hbm.at[v_v.at[off + j + nb]], abv[b], semv[b])

    return k(ab_tab, u2d, v2d)


def _sc_scatter_add(data, idx2d, zeros, n_pad, row_off=0):
    ch = idx2d.shape[1]
    n_chunks = idx2d.shape[0] // NW
    nb = min(2, n_chunks)
    rpt = n_pad // NS
    mesh = plsc.VectorSubcoreMesh(core_axis_name="c", subcore_axis_name="s")

    @functools.partial(
        pl.kernel,
        out_type=jax.ShapeDtypeStruct((NC, n_pad, D), jnp.float32),
        mesh=mesh,
        scratch_types=(
            [pltpu.VMEM((n_chunks, ch), jnp.int32)]
            + [pltpu.VMEM((ch, D), jnp.float32) for _ in range(nb)]
            + [pltpu.VMEM_SHARED((n_pad, D), jnp.float32)]
            + [pltpu.SemaphoreType.DMA for _ in range(2 * nb)]
        ),
    )
    def k(data_hbm, idx_hbm, zeros_hbm, out_hbm, idx_v, *rest):
        bufs = rest[:nb]
        agg_s = rest[nb]
        semi, semo = rest[nb + 1:2 * nb + 1], rest[2 * nb + 1:3 * nb + 1]
        cid = lax.axis_index("c")
        sid = lax.axis_index("s")
        wid = sid * NC + cid
        base_c = wid * n_chunks
        my = pl.ds(sid * rpt, rpt)
        pltpu.sync_copy(zeros_hbm, agg_s.at[my])
        pltpu.sync_copy(idx_hbm.at[pl.ds(base_c, n_chunks)], idx_v)
        plsc.subcore_barrier()

        def in_cp(j, b, sem):
            return pltpu.make_async_copy(
                data_hbm.at[pl.ds(row_off + (base_c + j) * ch, ch)],
                bufs[b], sem)

        for b in range(nb):
            in_cp(b, b, semi[b]).start()

        @pl.loop(0, n_chunks, step=nb)
        def _(g):
            for b in range(nb):
                j = g + b
                in_cp(j, b, semi[b]).wait()
                pltpu.async_copy(bufs[b], agg_s.at[idx_v.at[j]], semo[b],
                                 add=True)
            for b in range(nb):
                j = g + b
                pltpu.make_async_copy(
                    bufs[b], agg_s.at[idx_v.at[j]], semo[b]).wait()

                @pl.when(j + nb < n_chunks)
                def _():
                    in_cp(j + nb, b, semi[b]).start()

        plsc.subcore_barrier()
        pltpu.sync_copy(agg_s.at[my], out_hbm.at[cid, my])

    return k(data, idx2d, zeros)


def _tc_node_ab(f_pad, w1, w2, b1):
    np_, _ = f_pad.shape
    blk = np_ // 16

    def body(f_ref, w1_ref, w2_ref, b1_ref, ab_ref):
        f = f_ref[...]
        ab_ref[:, :D] = f @ w1_ref[...] + b1_ref[...]
        ab_ref[:, D:] = f @ w2_ref[...]

    return pl.pallas_call(
        body,
        grid=(16,),
        in_specs=[
            pl.BlockSpec((blk, D), lambda i: (i, 0)),
            pl.BlockSpec((D, D), lambda i: (0, 0)),
            pl.BlockSpec((D, D), lambda i: (0, 0)),
            pl.BlockSpec((1, D), lambda i: (0, 0)),
        ],
        out_specs=pl.BlockSpec((blk, 2 * D), lambda i: (i, 0)),
        out_shape=jax.ShapeDtypeStruct((np_, 2 * D), jnp.float32),
    )(f_pad, w1, w2, b1.reshape(1, D))


def _tc_relu(x):
    n = x.shape[0]
    blk = 4096
    grid = n // blk

    def body(x_ref, o_ref):
        o_ref[...] = jnp.maximum(x_ref[...], 0.0)

    return pl.pallas_call(
        body,
        grid=(grid,),
        in_specs=[pl.BlockSpec((blk, D), lambda i: (i, 0))],
        out_specs=pl.BlockSpec((blk, D), lambda i: (i, 0)),
        out_shape=jax.ShapeDtypeStruct((n, D), jnp.float32),
    )(x)


def _tc_combine(partials):
    np_ = partials.shape[1]
    blk = np_ // 16

    def body(p_ref, o_ref):
        o_ref[...] = p_ref[0] + p_ref[1]

    return pl.pallas_call(
        body,
        grid=(16,),
        in_specs=[pl.BlockSpec((2, blk, D), lambda i: (0, i, 0))],
        out_specs=pl.BlockSpec((blk, D), lambda i: (i, 0)),
        out_shape=jax.ShapeDtypeStruct((np_, D), jnp.float32),
    )(partials)


def _tc_msg_update(gathered, msg_old, pre, w3, blk_off=0):
    ep = msg_old.shape[0]
    blk = 1024
    nb = ep // blk
    nbh = nb // 2

    def body(g_ref, mr_ref, p_ref, w3_ref, o_ref):
        s = g_ref[...] - mr_ref[...]
        o_ref[...] = jnp.maximum(p_ref[...] + s @ w3_ref[...], 0.0)

    return pl.pallas_call(
        body,
        grid=(nb,),
        in_specs=[
            pl.BlockSpec((blk, D), lambda i: (blk_off + i, 0)),
            pl.BlockSpec((blk, D), lambda i: ((i + nbh) % nb, 0)),
            pl.BlockSpec((blk, D), lambda i: (i, 0)),
            pl.BlockSpec((D, D), lambda i: (0, 0)),
        ],
        out_specs=pl.BlockSpec((blk, D), lambda i: (i, 0)),
        out_shape=jax.ShapeDtypeStruct((ep, D), jnp.float32),
    )(gathered, msg_old, pre, w3)


def _tc_embw(emb, w1, w2, b1):
    def body(e_ref, w1_ref, w2_ref, b1_ref, o_ref):
        e = e_ref[...]
        o_ref[:, :D] = e @ w1_ref[...] + b1_ref[...]
        o_ref[:, D:2 * D] = e @ w2_ref[...]
        o_ref[:, 2 * D:] = e

    return pl.pallas_call(
        body,
        grid=(1,),
        in_specs=[
            pl.BlockSpec((VOCAB, D), lambda i: (0, 0)),
            pl.BlockSpec((D, D), lambda i: (0, 0)),
            pl.BlockSpec((D, D), lambda i: (0, 0)),
            pl.BlockSpec((1, D), lambda i: (0, 0)),
        ],
        out_specs=pl.BlockSpec((VOCAB, 3 * D), lambda i: (0, 0)),
        out_shape=jax.ShapeDtypeStruct((VOCAB, 3 * D), jnp.float32),
    )(emb, w1, w2, b1.reshape(1, D))


def _tc_onehot_rows(idx_col, table):
    n, w = idx_col.shape[0], table.shape[1]
    v = table.shape[0]
    blk = 640
    grid = n // blk

    def body(i_ref, t_ref, o_ref):
        ids = i_ref[...]
        io = lax.broadcasted_iota(jnp.int32, (1, v), 1)
        m = (ids == io).astype(jnp.float32)
        o_ref[...] = m @ t_ref[...]

    return pl.pallas_call(
        body,
        grid=(grid,),
        in_specs=[
            pl.BlockSpec((blk, 1), lambda i: (i, 0)),
            pl.BlockSpec((v, w), lambda i: (0, 0)),
        ],
        out_specs=pl.BlockSpec((blk, w), lambda i: (i, 0)),
        out_shape=jax.ShapeDtypeStruct((n, w), jnp.float32),
    )(idx_col, table)


def _tc_scatter_onehot(msg, dst3, n_pad):
    ep = msg.shape[0]
    eblk = 2048
    nblk = 640
    ne, nn = ep // eblk, n_pad // nblk

    def body(d_ref, m_ref, o_ref):
        n_i = pl.program_id(0)
        e_i = pl.program_id(1)
        ids = n_i * nblk + lax.broadcasted_iota(jnp.int32, (nblk, 1), 0)
        mask = (ids == d_ref[0]).astype(jnp.float32)
        contrib = mask @ m_ref[...]

        @pl.when(e_i == 0)
        def _():
            o_ref[...] = contrib

        @pl.when(e_i > 0)
        def _():
            o_ref[...] += contrib

    return pl.pallas_call(
        body,
        grid=(nn, ne),
        in_specs=[
            pl.BlockSpec((1, 1, eblk), lambda n, e: (e, 0, 0)),
            pl.BlockSpec((eblk, D), lambda n, e: (e, 0)),
        ],
        out_specs=pl.BlockSpec((nblk, D), lambda n, e: (n, 0)),
        out_shape=jax.ShapeDtypeStruct((n_pad, D), jnp.float32),
    )(dst3, msg)


def _tc_combine2(partials_g, agg_t):
    npg, npt = partials_g.shape[1], agg_t.shape[0]
    blk = 128
    nbg, nbt = npg // blk, npt // blk

    def body(p_ref, t_ref, o_ref):
        i = pl.program_id(0)

        @pl.when(i < nbg)
        def _():
            o_ref[...] = p_ref[0] + p_ref[1]

        @pl.when(i >= nbg)
        def _():
            o_ref[...] = t_ref[...]

    return pl.pallas_call(
        body,
        grid=(nbg + nbt,),
        in_specs=[
            pl.BlockSpec((2, blk, D),
                         lambda i: (0, jnp.minimum(i, nbg - 1), 0)),
            pl.BlockSpec((blk, D), lambda i: (jnp.maximum(i - nbg, 0), 0)),
        ],
        out_specs=pl.BlockSpec((blk, D), lambda i: (i, 0)),
        out_shape=jax.ShapeDtypeStruct((npg + npt, D), jnp.float32),
    )(partials_g, agg_t)


def _tc_readout(f, ns, u1, u2, b2):
    n = f.shape[0]
    blk = 1000
    grid = n // blk

    def body(f_ref, ns_ref, u1_ref, u2_ref, b2_ref, o_ref):
        o_ref[...] = jnp.maximum(
            f_ref[...] @ u1_ref[...] + ns_ref[...] @ u2_ref[...] + b2_ref[...],
            0.0)

    return pl.pallas_call(
        body,
        grid=(grid,),
        in_specs=[
            pl.BlockSpec((blk, D), lambda i: (i, 0)),
            pl.BlockSpec((blk, D), lambda i: (i, 0)),
            pl.BlockSpec((D, D), lambda i: (0, 0)),
            pl.BlockSpec((D, D), lambda i: (0, 0)),
            pl.BlockSpec((1, D), lambda i: (0, 0)),
        ],
        out_specs=pl.BlockSpec((blk, D), lambda i: (i, 0)),
        out_shape=jax.ShapeDtypeStruct((n, D), jnp.float32),
    )(f, ns, u1, u2, b2.reshape(1, D))


EHP_G = _round_up(EH_G, NW * 128)
EP_G = 2 * EHP_G
NP_G = _round_up(N_G + 1, NS * 8)
EHP_T = _round_up(EH_T, NW * 128)
EP_T = 2 * EHP_T
NP_T = _round_up(N_T + 1, NS * 8)
EP_CAT = EP_G + EP_T


def kernel(f_G, u_G, v_G, id_T, u_T, v_T, embeddings, W1_G, W2_G, W3_G, b1_G,
           U1_G, U2_G, b2_G, W1_T, W2_T, W3_T, b1_T, U1_T, U2_T, b2_T):
    u_gp = _pad_to(u_G, EHP_G, N_G)
    v_gp = _pad_to(v_G, EHP_G, N_G)
    u_tp = _pad_to(u_T, EHP_T, N_T)
    v_tp = _pad_to(v_T, EHP_T, N_T)
    dst_g = jnp.concatenate([v_gp, u_gp]).reshape(EP_G // 64, 64)
    dst_t3 = jnp.concatenate([v_tp, u_tp]).reshape(EP_T // 2048, 1, 2048)
    src_cat = jnp.concatenate(
        [u_gp, v_gp, NP_G + u_tp, NP_G + v_tp]).reshape(EP_CAT // 128, 128)
    u2d_g = u_gp.reshape(EHP_G // 32, 32)
    v2d_g = v_gp.reshape(EHP_G // 32, 32)
    u2d_t = u_tp.reshape(EHP_T // 32, 32)
    v2d_t = v_tp.reshape(EHP_T // 32, 32)
    id_col = _pad_to(id_T, NP_T).reshape(NP_T, 1)
    f_gpad = _pad_to(f_G, NP_G)
    zeros_g = jnp.zeros((NP_G // NS, D), jnp.float32)

    ab_g = _tc_node_ab(f_gpad, W1_G, W2_G, b1_G)
    pre_g = _sc_pre_gather(ab_g, u2d_g, v2d_g, EHP_G, frac0=0.6)
    embt = _tc_embw(embeddings, W1_T, W2_T, b1_T)
    fab_t = _tc_onehot_rows(id_col, embt)
    ab_t = fab_t[:, :2 * D]
    f_t = fab_t[:N_T, 2 * D:]
    pre_t = _sc_pre_gather(ab_t, u2d_t, v2d_t, EHP_T)

    msg_g = _tc_relu(pre_g)
    msg_t = _tc_relu(pre_t)
    for _ in range(N_ITERS - 1):
        partials_g = _sc_scatter_add(msg_g, dst_g, zeros_g, NP_G)
        agg_t = _tc_scatter_onehot(msg_t, dst_t3, NP_T)
        table = _tc_combine2(partials_g, agg_t)
        gathered = _sc_gather(table, src_cat, EP_CAT, frac0=0.61)
        msg_g = _tc_msg_update(gathered, msg_g, pre_g, W3_G)
        msg_t = _tc_msg_update(gathered, msg_t, pre_t, W3_T,
                               blk_off=EP_G // 1024)

    partials_g = _sc_scatter_add(msg_g, dst_g, zeros_g, NP_G)
    ns_g = _tc_combine(partials_g)[:N_G]
    ns_t = _tc_scatter_onehot(msg_t, dst_t3, NP_T)[:N_T]
    x_G = _tc_readout(f_G, ns_g, U1_G, U2_G, b2_G)
    x_T = _tc_readout(f_t, ns_t, U1_T, U2_T, b2_T)
    return x_G, x_T
